# Initial kernel scaffold; baseline (speedup 1.0000x reference)
import jax
import jax.numpy as jnp
from jax.experimental import pallas as pl


def kernel(log_alpha_0, log_alpha_1):
    raise NotImplementedError("write your pallas kernel here")


# speedup vs baseline: 4.3970x; 4.3970x over previous
import jax
import jax.numpy as jnp
from jax.experimental import pallas as pl
from jax.experimental.pallas import tpu as pltpu

_N = 4096
_SH = 256
_NSTRIP = _N // _SH
_N_ITER = 10


def _sinkhorn_body(a_hbm, out_hbm, kmat, buf, u, v, in_sems, out_sems):
    def _in_copy(s, slot):
        return pltpu.make_async_copy(
            a_hbm.at[pl.ds(s * _SH, _SH), :], buf.at[slot], in_sems.at[slot])

    _in_copy(0, 0).start()

    def _phase0(s, carry):
        slot = jax.lax.rem(s, 2)

        @pl.when(s + 1 < _NSTRIP)
        def _():
            _in_copy(s + 1, 1 - slot).start()

        _in_copy(s, slot).wait()
        kmat[pl.ds(s * _SH, _SH), :] = jnp.exp(buf[slot]).astype(jnp.bfloat16)
        return carry

    jax.lax.fori_loop(0, _NSTRIP, _phase0, 0)

    v[...] = jnp.ones((1, _N), jnp.float32)

    def _one_iter(t, carry):
        def _row_strip(s, c):
            k = kmat[pl.ds(s * _SH, _SH), :].astype(jnp.float32)
            rs = jnp.sum(k * v[...], axis=1, keepdims=True)
            u[pl.ds(s * _SH, _SH), :] = 1.0 / rs
            return c

        jax.lax.fori_loop(0, _NSTRIP, _row_strip, 0)

        def _col_strip(s, acc):
            k = kmat[pl.ds(s * _SH, _SH), :].astype(jnp.float32)
            return acc + jnp.sum(k * u[pl.ds(s * _SH, _SH), :], axis=0,
                                 keepdims=True)

        csum = jax.lax.fori_loop(0, _NSTRIP, _col_strip,
                                 jnp.zeros((1, _N), jnp.float32))
        v[...] = 1.0 / csum
        return carry

    jax.lax.fori_loop(0, _N_ITER, _one_iter, 0)

    def _out_copy(s, slot):
        return pltpu.make_async_copy(
            buf.at[slot], out_hbm.at[pl.ds(s * _SH, _SH), :],
            out_sems.at[slot])

    def _out_phase(s, carry):
        slot = jax.lax.rem(s, 2)

        @pl.when(s >= 2)
        def _():
            _out_copy(s - 2, slot).wait()

        k = kmat[pl.ds(s * _SH, _SH), :].astype(jnp.float32)
        buf[slot] = k * u[pl.ds(s * _SH, _SH), :] * v[...]
        _out_copy(s, slot).start()
        return carry

    jax.lax.fori_loop(0, _NSTRIP, _out_phase, 0)
    _out_copy(_NSTRIP - 2, 0).wait()
    _out_copy(_NSTRIP - 1, 1).wait()


def _sinkhorn(a):
    return pl.pallas_call(
        _sinkhorn_body,
        out_shape=jax.ShapeDtypeStruct((_N, _N), jnp.float32),
        in_specs=[pl.BlockSpec(memory_space=pl.ANY)],
        out_specs=pl.BlockSpec(memory_space=pl.ANY),
        scratch_shapes=[
            pltpu.VMEM((_N, _N), jnp.bfloat16),
            pltpu.VMEM((2, _SH, _N), jnp.float32),
            pltpu.VMEM((_N, 1), jnp.float32),
            pltpu.VMEM((1, _N), jnp.float32),
            pltpu.SemaphoreType.DMA((2,)),
            pltpu.SemaphoreType.DMA((2,)),
        ],
        compiler_params=pltpu.CompilerParams(
            vmem_limit_bytes=63 * 1024 * 1024,
        ),
    )(a)


def kernel(log_alpha_0, log_alpha_1):
    return _sinkhorn(log_alpha_0), _sinkhorn(log_alpha_1)
